# baseline (speedup 1.0000x reference)
import jax
import jax.numpy as jnp
from jax.experimental import pallas as pl

DIM = 4096
NUM_EXPERTS = 64
HEAD = 4
H = NUM_EXPERTS * HEAD
TOP_K = 8
TOKENS = 4 * 2048
BT = 512


def _tc_body(x_ref, wg_ref, bg_ref, wp_ref, bp_ref, scores_ref, routed_ref):
    h = jnp.dot(x_ref[...], wg_ref[...], preferred_element_type=jnp.float32)
    h = h + bg_ref[...]
    h = 0.5 * h * (1.0 + jax.lax.erf(h * 0.7071067811865476))
    logits = jnp.dot(h, wp_ref[...], preferred_element_type=jnp.float32)
    logits = logits + bp_ref[...]

    lt = logits.T
    m = jnp.max(lt, axis=0, keepdims=True)
    e = jnp.exp(lt - m)
    sT = e / jnp.sum(e, axis=0, keepdims=True)
    scores_ref[...] = sT.T

    work = sT
    cum = jnp.zeros((1, BT), jnp.float32)
    t8 = jnp.full((1, BT), -1.0, jnp.float32)
    crossed = jnp.zeros((1, BT), jnp.bool_)
    for _ in range(TOP_K):
        cur = jnp.max(work, axis=0, keepdims=True)
        eq = work == cur
        cnt = jnp.sum(jnp.where(eq, 1.0, 0.0), axis=0, keepdims=True)
        newcum = cum + cnt
        now = jnp.logical_and(newcum >= float(TOP_K), jnp.logical_not(crossed))
        t8 = jnp.where(now, cur, t8)
        crossed = jnp.logical_or(crossed, now)
        work = jnp.where(eq, -1.0, work)
        cum = newcum

    gt = sT > t8
    eqm = sT == t8
    eqf = jnp.where(eqm, 1.0, 0.0)
    r_idx = jax.lax.broadcasted_iota(jnp.int32, (NUM_EXPERTS, NUM_EXPERTS), 0)
    c_idx = jax.lax.broadcasted_iota(jnp.int32, (NUM_EXPERTS, NUM_EXPERTS), 1)
    ltri = jnp.where(c_idx <= r_idx, 1.0, 0.0).astype(jnp.bfloat16)
    prefix = jnp.dot(ltri, eqf.astype(jnp.bfloat16),
                     preferred_element_type=jnp.float32)
    cntgt = jnp.sum(jnp.where(gt, 1.0, 0.0), axis=0, keepdims=True)
    need = float(TOP_K) - cntgt
    mask = jnp.logical_or(gt, jnp.logical_and(eqm, prefix <= need))
    routed_ref[...] = jnp.where(mask, 1.0, 0.0).T


@jax.jit
def _gate(x2d, W_gate, b_gate, W_proj, b_proj):
    grid = TOKENS // BT
    return pl.pallas_call(
        _tc_body,
        grid=(grid,),
        in_specs=[
            pl.BlockSpec((BT, DIM), lambda i: (i, 0)),
            pl.BlockSpec((DIM, H), lambda i: (0, 0)),
            pl.BlockSpec((1, H), lambda i: (0, 0)),
            pl.BlockSpec((H, NUM_EXPERTS), lambda i: (0, 0)),
            pl.BlockSpec((1, NUM_EXPERTS), lambda i: (0, 0)),
        ],
        out_specs=[
            pl.BlockSpec((BT, NUM_EXPERTS), lambda i: (i, 0)),
            pl.BlockSpec((BT, NUM_EXPERTS), lambda i: (i, 0)),
        ],
        out_shape=[
            jax.ShapeDtypeStruct((TOKENS, NUM_EXPERTS), jnp.float32),
            jax.ShapeDtypeStruct((TOKENS, NUM_EXPERTS), jnp.float32),
        ],
    )(x2d, W_gate, b_gate.reshape(1, H), W_proj, b_proj.reshape(1, NUM_EXPERTS))


def kernel(x, W_gate, b_gate, W_proj, b_proj):
    B, T, _ = x.shape
    scores, routed = _gate(x.reshape(B * T, DIM), W_gate, b_gate, W_proj, b_proj)
    gate_scores = scores.reshape(B, T, NUM_EXPERTS)
    routed_experts = routed.reshape(B, T, NUM_EXPERTS)
    return (gate_scores, routed_experts, jnp.float32(0.0))


# speedup vs baseline: 1.0616x; 1.0616x over previous
import jax
import jax.numpy as jnp
from jax.experimental import pallas as pl

DIM = 4096
NUM_EXPERTS = 64
HEAD = 4
H = NUM_EXPERTS * HEAD
TOP_K = 8
TOKENS = 4 * 2048
BT = 1024


def _tc_body(x_ref, wg_ref, bg_ref, wp_ref, bp_ref, scores_ref, routed_ref):
    h = jnp.dot(x_ref[...], wg_ref[...], preferred_element_type=jnp.float32)
    h = h + bg_ref[...]
    h = 0.5 * h * (1.0 + jax.lax.erf(h * 0.7071067811865476))
    logits = jnp.dot(h, wp_ref[...], preferred_element_type=jnp.float32)
    logits = logits + bp_ref[...]

    lt = logits.T
    m = jnp.max(lt, axis=0, keepdims=True)
    e = jnp.exp(lt - m)
    sT = e / jnp.sum(e, axis=0, keepdims=True)
    scores_ref[...] = sT.T

    work = sT
    cum = jnp.zeros((1, BT), jnp.float32)
    t8 = jnp.full((1, BT), -1.0, jnp.float32)
    crossed = jnp.zeros((1, BT), jnp.bool_)
    for _ in range(TOP_K):
        cur = jnp.max(work, axis=0, keepdims=True)
        eq = work == cur
        cnt = jnp.sum(jnp.where(eq, 1.0, 0.0), axis=0, keepdims=True)
        newcum = cum + cnt
        now = jnp.logical_and(newcum >= float(TOP_K), jnp.logical_not(crossed))
        t8 = jnp.where(now, cur, t8)
        crossed = jnp.logical_or(crossed, now)
        work = jnp.where(eq, -1.0, work)
        cum = newcum

    gt = sT > t8
    eqm = sT == t8
    eqf = jnp.where(eqm, 1.0, 0.0)
    r_idx = jax.lax.broadcasted_iota(jnp.int32, (NUM_EXPERTS, NUM_EXPERTS), 0)
    c_idx = jax.lax.broadcasted_iota(jnp.int32, (NUM_EXPERTS, NUM_EXPERTS), 1)
    ltri = jnp.where(c_idx <= r_idx, 1.0, 0.0).astype(jnp.bfloat16)
    prefix = jnp.dot(ltri, eqf.astype(jnp.bfloat16),
                     preferred_element_type=jnp.float32)
    cntgt = jnp.sum(jnp.where(gt, 1.0, 0.0), axis=0, keepdims=True)
    need = float(TOP_K) - cntgt
    mask = jnp.logical_or(gt, jnp.logical_and(eqm, prefix <= need))
    routed_ref[...] = jnp.where(mask, 1.0, 0.0).T


@jax.jit
def _gate(x2d, W_gate, b_gate, W_proj, b_proj):
    grid = TOKENS // BT
    return pl.pallas_call(
        _tc_body,
        grid=(grid,),
        in_specs=[
            pl.BlockSpec((BT, DIM), lambda i: (i, 0)),
            pl.BlockSpec((DIM, H), lambda i: (0, 0)),
            pl.BlockSpec((1, H), lambda i: (0, 0)),
            pl.BlockSpec((H, NUM_EXPERTS), lambda i: (0, 0)),
            pl.BlockSpec((1, NUM_EXPERTS), lambda i: (0, 0)),
        ],
        out_specs=[
            pl.BlockSpec((BT, NUM_EXPERTS), lambda i: (i, 0)),
            pl.BlockSpec((BT, NUM_EXPERTS), lambda i: (i, 0)),
        ],
        out_shape=[
            jax.ShapeDtypeStruct((TOKENS, NUM_EXPERTS), jnp.float32),
            jax.ShapeDtypeStruct((TOKENS, NUM_EXPERTS), jnp.float32),
        ],
    )(x2d, W_gate, b_gate.reshape(1, H), W_proj, b_proj.reshape(1, NUM_EXPERTS))


def kernel(x, W_gate, b_gate, W_proj, b_proj):
    B, T, _ = x.shape
    scores, routed = _gate(x.reshape(B * T, DIM), W_gate, b_gate, W_proj, b_proj)
    gate_scores = scores.reshape(B, T, NUM_EXPERTS)
    routed_experts = routed.reshape(B, T, NUM_EXPERTS)
    return (gate_scores, routed_experts, jnp.float32(0.0))
